# baseline (speedup 1.0000x reference)
import jax
import jax.numpy as jnp
from jax.experimental import pallas as pl
from jax.experimental.pallas import tpu as pltpu

N = 10000
D = 128
BM = 1024
BK = 1024
NI = (N + BM - 1) // BM
NK = (N + BK - 1) // BK
NKP = NK // 2
NPAD = NI * BM


def _body(A0_ref, A1_ref, At0_ref, At1_ref, xs_ref, xt_ref, ws_ref, wt_ref,
          o_ref, acc_ref, ys_ref, yt_ref, xps_ref, xpt_ref):
    p = pl.program_id(0)
    m = pl.program_id(1)
    i = pl.program_id(2)
    k = pl.program_id(3)

    @pl.when(jnp.logical_and(jnp.logical_and(p == 0, m == 0),
                             jnp.logical_and(i == 0, k == 0)))
    def _():
        xps_ref[0:N, :] = xs_ref[...]
        xps_ref[N:NPAD, :] = jnp.zeros((NPAD - N, D), jnp.float32)
        xpt_ref[0:N, :] = xt_ref[...]
        xpt_ref[N:NPAD, :] = jnp.zeros((NPAD - N, D), jnp.float32)

    def steps(a0_ref, a1_ref, rhs_fn, epilogue):
        def pair(masked):
            d0 = jnp.dot(a0_ref[...], rhs_fn(2 * k),
                         preferred_element_type=jnp.float32)
            if masked:
                rem_k = N - (NK - 1) * BK
                colmask = jax.lax.broadcasted_iota(
                    jnp.int32, (BM, BK), 1) < rem_k
                a1 = jnp.where(colmask, a1_ref[...], 0.0)
            else:
                a1 = a1_ref[...]
            return d0 + jnp.dot(a1, rhs_fn(2 * k + 1),
                                preferred_element_type=jnp.float32)

        @pl.when(k == 0)
        def _():
            acc_ref[...] = pair(False)

        @pl.when(jnp.logical_and(k > 0, k < NKP - 1))
        def _():
            acc_ref[...] += pair(False)

        @pl.when(k == NKP - 1)
        def _():
            epilogue(acc_ref[...] + pair(True))

    def hop1(a0_ref, a1_ref, x_ref, y_ref):
        def rhs_fn(kk):
            return x_ref[pl.ds(kk * BK, BK), :]

        def epilogue(acc):
            rem_m = N - i * BM
            rowmask = jax.lax.broadcasted_iota(jnp.int32, (BM, D), 0) < rem_m
            y_ref[pl.ds(i * BM, BM), :] = jnp.where(rowmask, acc, 0.0)

        steps(a0_ref, a1_ref, rhs_fn, epilogue)

    def hop2(a0_ref, a1_ref, x_ref, y_ref, w_ref):
        def rhs_fn(kk):
            return (w_ref[1, 0] * x_ref[pl.ds(kk * BK, BK), :]
                    + w_ref[2, 0] * y_ref[pl.ds(kk * BK, BK), :])

        def epilogue(acc):
            o_ref[...] = w_ref[0, 0] * x_ref[pl.ds(i * BM, BM), :] + acc

        steps(a0_ref, a1_ref, rhs_fn, epilogue)

    @pl.when(jnp.logical_and(p == 0, m == 0))
    def _():
        hop1(A0_ref, A1_ref, xps_ref, ys_ref)

    @pl.when(jnp.logical_and(p == 0, m == 1))
    def _():
        hop1(At0_ref, At1_ref, xpt_ref, yt_ref)

    @pl.when(jnp.logical_and(p == 1, m == 0))
    def _():
        hop2(A0_ref, A1_ref, xps_ref, ys_ref, ws_ref)

    @pl.when(jnp.logical_and(p == 1, m == 1))
    def _():
        hop2(At0_ref, At1_ref, xpt_ref, yt_ref, wt_ref)


def _feat(x_s, x_t, A, At, w_s, w_t, interpret=False):
    return pl.pallas_call(
        _body,
        grid=(2, 2, NI, NKP),
        in_specs=[
            pl.BlockSpec((BM, BK),
                         lambda p, m, i, k: (jnp.where(m == 0, i, 0),
                                             jnp.where(m == 0, 2 * k, 0))),
            pl.BlockSpec((BM, BK),
                         lambda p, m, i, k: (jnp.where(m == 0, i, 0),
                                             jnp.where(m == 0, 2 * k + 1, 1))),
            pl.BlockSpec((BM, BK),
                         lambda p, m, i, k: (jnp.where(m == 1, i, 0),
                                             jnp.where(m == 1, 2 * k, 0))),
            pl.BlockSpec((BM, BK),
                         lambda p, m, i, k: (jnp.where(m == 1, i, 0),
                                             jnp.where(m == 1, 2 * k + 1, 1))),
            pl.BlockSpec((N, D), lambda p, m, i, k: (0, 0)),
            pl.BlockSpec((N, D), lambda p, m, i, k: (0, 0)),
            pl.BlockSpec(memory_space=pltpu.SMEM),
            pl.BlockSpec(memory_space=pltpu.SMEM),
        ],
        out_specs=pl.BlockSpec((BM, D),
                               lambda p, m, i, k: (jnp.where(p == 0, 0, i), m)),
        out_shape=jax.ShapeDtypeStruct((N, 2 * D), jnp.float32),
        scratch_shapes=[
            pltpu.VMEM((BM, D), jnp.float32),
            pltpu.VMEM((NPAD, D), jnp.float32),
            pltpu.VMEM((NPAD, D), jnp.float32),
            pltpu.VMEM((NPAD, D), jnp.float32),
            pltpu.VMEM((NPAD, D), jnp.float32),
        ],
        compiler_params=pltpu.CompilerParams(
            dimension_semantics=("arbitrary",) * 4,
            vmem_limit_bytes=100 * 1024 * 1024),
        interpret=interpret,
    )(A, A, At, At, x_s, x_t, w_s, w_t)


def kernel(x_s, x_t, A, At, w_s, w_t):
    return _feat(x_s, x_t, A, At, w_s, w_t)


# speedup vs baseline: 1.0046x; 1.0046x over previous
import jax
import jax.numpy as jnp
from jax.experimental import pallas as pl
from jax.experimental.pallas import tpu as pltpu

N = 10000
D = 128
BM = 1024
BK = 2048
NI = (N + BM - 1) // BM
NK = (N + BK - 1) // BK
NPAD = NI * BM


def _body(A_ref, At_ref, xs_ref, xt_ref, ws_ref, wt_ref,
          o_ref, acc_ref, ys_ref, yt_ref, xps_ref, xpt_ref):
    p = pl.program_id(0)
    m = pl.program_id(1)
    i = pl.program_id(2)
    k = pl.program_id(3)

    @pl.when(jnp.logical_and(jnp.logical_and(p == 0, m == 0),
                             jnp.logical_and(i == 0, k == 0)))
    def _():
        xps_ref[0:N, :] = xs_ref[...]
        xps_ref[N:NPAD, :] = jnp.zeros((NPAD - N, D), jnp.float32)
        xpt_ref[0:N, :] = xt_ref[...]
        xpt_ref[N:NPAD, :] = jnp.zeros((NPAD - N, D), jnp.float32)

    def masked_a(a_ref):
        rem_k = N - k * BK
        colmask = jax.lax.broadcasted_iota(jnp.int32, (BM, BK), 1) < rem_k
        return jnp.where(colmask, a_ref[...], 0.0)

    def steps(a_ref, rhs_fn, epilogue):
        @pl.when(k == 0)
        def _():
            acc_ref[...] = jnp.dot(a_ref[...], rhs_fn(),
                                   preferred_element_type=jnp.float32)

        @pl.when(jnp.logical_and(k > 0, k < NK - 1))
        def _():
            acc_ref[...] += jnp.dot(a_ref[...], rhs_fn(),
                                    preferred_element_type=jnp.float32)

        @pl.when(k == NK - 1)
        def _():
            acc = acc_ref[...] + jnp.dot(masked_a(a_ref), rhs_fn(),
                                         preferred_element_type=jnp.float32)
            epilogue(acc)

    def hop1(a_ref, x_ref, y_ref):
        def rhs_fn():
            return x_ref[pl.ds(k * BK, BK), :]

        def epilogue(acc):
            rem_m = N - i * BM
            rowmask = jax.lax.broadcasted_iota(jnp.int32, (BM, D), 0) < rem_m
            y_ref[pl.ds(i * BM, BM), :] = jnp.where(rowmask, acc, 0.0)

        steps(a_ref, rhs_fn, epilogue)

    def hop2(a_ref, x_ref, y_ref, w_ref):
        def rhs_fn():
            return (w_ref[1, 0] * x_ref[pl.ds(k * BK, BK), :]
                    + w_ref[2, 0] * y_ref[pl.ds(k * BK, BK), :])

        def epilogue(acc):
            o_ref[...] = w_ref[0, 0] * x_ref[pl.ds(i * BM, BM), :] + acc

        steps(a_ref, rhs_fn, epilogue)

    @pl.when(jnp.logical_and(p == 0, m == 0))
    def _():
        hop1(A_ref, xps_ref, ys_ref)

    @pl.when(jnp.logical_and(p == 0, m == 1))
    def _():
        hop1(At_ref, xpt_ref, yt_ref)

    @pl.when(jnp.logical_and(p == 1, m == 0))
    def _():
        hop2(A_ref, xps_ref, ys_ref, ws_ref)

    @pl.when(jnp.logical_and(p == 1, m == 1))
    def _():
        hop2(At_ref, xpt_ref, yt_ref, wt_ref)


def _feat(x_s, x_t, A, At, w_s, w_t, interpret=False):
    return pl.pallas_call(
        _body,
        grid=(2, 2, NI, NK),
        in_specs=[
            pl.BlockSpec((BM, BK),
                         lambda p, m, i, k: (jnp.where(m == 0, i, 0),
                                             jnp.where(m == 0, k, 0))),
            pl.BlockSpec((BM, BK),
                         lambda p, m, i, k: (jnp.where(m == 1, i, 0),
                                             jnp.where(m == 1, k, 0))),
            pl.BlockSpec((N, D), lambda p, m, i, k: (0, 0)),
            pl.BlockSpec((N, D), lambda p, m, i, k: (0, 0)),
            pl.BlockSpec(memory_space=pltpu.SMEM),
            pl.BlockSpec(memory_space=pltpu.SMEM),
        ],
        out_specs=pl.BlockSpec((BM, D),
                               lambda p, m, i, k: (jnp.where(p == 0, 0, i), m)),
        out_shape=jax.ShapeDtypeStruct((N, 2 * D), jnp.float32),
        scratch_shapes=[
            pltpu.VMEM((BM, D), jnp.float32),
            pltpu.VMEM((NPAD, D), jnp.float32),
            pltpu.VMEM((NPAD, D), jnp.float32),
            pltpu.VMEM((NPAD, D), jnp.float32),
            pltpu.VMEM((NPAD, D), jnp.float32),
        ],
        compiler_params=pltpu.CompilerParams(
            dimension_semantics=("arbitrary",) * 4,
            vmem_limit_bytes=100 * 1024 * 1024),
        interpret=interpret,
    )(A, At, x_s, x_t, w_s, w_t)


def kernel(x_s, x_t, A, At, w_s, w_t):
    return _feat(x_s, x_t, A, At, w_s, w_t)
